# Initial kernel scaffold; baseline (speedup 1.0000x reference)
import jax
import jax.numpy as jnp
from jax.experimental import pallas as pl


def kernel(x):
    raise NotImplementedError("write your pallas kernel here")


# speedup vs baseline: 28.2544x; 28.2544x over previous
import functools

import jax
import jax.numpy as jnp
from jax.experimental import pallas as pl

B, N, F = 8, 50000, 128
CHUNK = 2000
NCHUNK = N // CHUNK


def _pool_body(x_ref, o_ref):
    j = pl.program_id(0)

    @pl.when(j == 0)
    def _init():
        o_ref[...] = jnp.zeros_like(o_ref)

    o_ref[...] += jnp.sum(x_ref[...], axis=1)

    @pl.when(j == NCHUNK - 1)
    def _finish():
        o_ref[...] = o_ref[...] * (1.0 / N)


@jax.jit
def kernel(x):
    return pl.pallas_call(
        _pool_body,
        grid=(NCHUNK,),
        in_specs=[pl.BlockSpec((B, CHUNK, F), lambda j: (0, j, 0))],
        out_specs=pl.BlockSpec((B, F), lambda j: (0, 0)),
        out_shape=jax.ShapeDtypeStruct((B, F), jnp.float32),
    )(x)
